# Initial kernel scaffold; baseline (speedup 1.0000x reference)
import jax
import jax.numpy as jnp
from jax.experimental import pallas as pl


def kernel(z, embed_weight):
    raise NotImplementedError("write your pallas kernel here")


# speedup vs baseline: 1.0506x; 1.0506x over previous
import functools

import jax
import jax.numpy as jnp
from jax import lax
from jax.experimental import pallas as pl
from jax.experimental.pallas import tpu as pltpu

N_EMBED = 8192
D = 256
BETA = 1.0
EPS = 1e-12

TN = 256
TK = 2048
N_TOKENS = 32768


def _normalize_embed_body(e_ref, en_out_ref, e_norm_ref):
    x = e_ref[...]
    n2 = jnp.sum(x * x, axis=1, keepdims=True)
    n = jnp.sqrt(n2)
    xn = x / jnp.maximum(n, EPS)
    e_norm_ref[...] = xn
    en_out_ref[...] = jnp.sum(xn * xn, axis=1, keepdims=True)


def _argmin_body(z_ref, e_norm_ref, en_ref, idx_ref, d2_ref, acc_ref):
    i = pl.program_id(0)
    z = z_ref[...]
    zsq = jnp.sum(z * z, axis=1, keepdims=True)
    zn = z / jnp.maximum(jnp.sqrt(zsq), EPS)
    zn2 = jnp.sum(zn * zn, axis=1, keepdims=True)

    best = jnp.full((TN, 1), jnp.inf, jnp.float32)
    bidx = jnp.zeros((TN, 1), jnp.int32)
    for kc in range(N_EMBED // TK):
        e_chunk = e_norm_ref[pl.ds(kc * TK, TK), :]
        en_chunk = en_ref[pl.ds(kc * TK, TK), :]
        s = lax.dot_general(
            zn, e_chunk,
            dimension_numbers=(((1,), (1,)), ((), ())),
            preferred_element_type=jnp.float32,
        )
        a = zn2 + en_chunk[:, 0][None, :]
        d2 = a - 2.0 * s
        m = jnp.min(d2, axis=1, keepdims=True)
        iota = lax.broadcasted_iota(jnp.int32, d2.shape, 1)
        li = jnp.min(jnp.where(d2 == m, iota, N_EMBED), axis=1, keepdims=True)
        upd = m < best
        bidx = jnp.where(upd, li + kc * TK, bidx)
        best = jnp.where(upd, m, best)

    idx_ref[...] = bidx
    d2_ref[...] = best

    @pl.when(i == 0)
    def _():
        acc_ref[...] = jnp.zeros((1, 1), jnp.float32)

    acc_ref[...] += jnp.sum(best, keepdims=True)


@jax.jit
def kernel(z, embed_weight):
    Bs, Ts, Ds = z.shape
    z_flat = z.reshape(-1, Ds)

    en, e_norm = pl.pallas_call(
        _normalize_embed_body,
        out_shape=(
            jax.ShapeDtypeStruct((N_EMBED, 1), jnp.float32),
            jax.ShapeDtypeStruct((N_EMBED, D), jnp.float32),
        ),
    )(embed_weight)

    grid = (N_TOKENS // TN,)
    idx, d2min, acc = pl.pallas_call(
        _argmin_body,
        grid=grid,
        in_specs=[
            pl.BlockSpec((TN, D), lambda i: (i, 0)),
            pl.BlockSpec((N_EMBED, D), lambda i: (0, 0)),
            pl.BlockSpec((N_EMBED, 1), lambda i: (0, 0)),
        ],
        out_specs=(
            pl.BlockSpec((TN, 1), lambda i: (i, 0)),
            pl.BlockSpec((TN, 1), lambda i: (i, 0)),
            pl.BlockSpec((1, 1), lambda i: (0, 0)),
        ),
        out_shape=(
            jax.ShapeDtypeStruct((N_TOKENS, 1), jnp.int32),
            jax.ShapeDtypeStruct((N_TOKENS, 1), jnp.float32),
            jax.ShapeDtypeStruct((1, 1), jnp.float32),
        ),
    )(z_flat, e_norm, en)

    indices = idx[:, 0]
    quantized = jnp.take(e_norm, indices, axis=0)
    loss = acc[0, 0] * ((1.0 + BETA) / (N_TOKENS * D))
    return (quantized.reshape(Bs, Ts, Ds), loss, indices.reshape(Bs, Ts))


# speedup vs baseline: 1.2029x; 1.1449x over previous
import functools

import jax
import jax.numpy as jnp
from jax import lax
from jax.experimental import pallas as pl
from jax.experimental.pallas import tpu as pltpu

N_EMBED = 8192
D = 256
BETA = 1.0
EPS = 1e-12

TN = 256
TK = 2048
N_TOKENS = 32768


def _normalize_embed_body(e_ref, en_out_ref, e_norm_ref):
    x = e_ref[...]
    n2 = jnp.sum(x * x, axis=1, keepdims=True)
    n = jnp.sqrt(n2)
    xn = x / jnp.maximum(n, EPS)
    e_norm_ref[...] = xn
    en_out_ref[...] = jnp.sum(xn * xn, axis=1, keepdims=True).reshape(1, N_EMBED)


def _argmin_body(z_ref, e_norm_ref, en_ref, idx_ref, d2_ref, acc_ref):
    i = pl.program_id(0)
    z = z_ref[...]
    zsq = jnp.sum(z * z, axis=1, keepdims=True)
    zn = z / jnp.maximum(jnp.sqrt(zsq), EPS)
    zn2 = jnp.sum(zn * zn, axis=1, keepdims=True)
    zm = -2.0 * zn

    best = jnp.full((TN, 1), jnp.inf, jnp.float32)
    bidx = jnp.zeros((TN, 1), jnp.int32)
    for kc in range(N_EMBED // TK):
        e_chunk = e_norm_ref[pl.ds(kc * TK, TK), :]
        en_chunk = en_ref[:, pl.ds(kc * TK, TK)]
        s2 = lax.dot_general(
            zm, e_chunk,
            dimension_numbers=(((1,), (1,)), ((), ())),
            preferred_element_type=jnp.float32,
        )
        a = zn2 + en_chunk
        d2 = a + s2
        m = jnp.min(d2, axis=1, keepdims=True)
        iota = lax.broadcasted_iota(jnp.int32, d2.shape, 1)
        li = jnp.min(jnp.where(d2 == m, iota, N_EMBED), axis=1, keepdims=True)
        upd = m < best
        bidx = jnp.where(upd, li + kc * TK, bidx)
        best = jnp.where(upd, m, best)

    idx_ref[...] = bidx.reshape(1, 1, TN)
    d2_ref[...] = best.reshape(1, 1, TN)

    @pl.when(i == 0)
    def _():
        acc_ref[...] = jnp.zeros((1, 1), jnp.float32)

    acc_ref[...] += jnp.sum(best, keepdims=True)


@jax.jit
def kernel(z, embed_weight):
    Bs, Ts, Ds = z.shape
    z_flat = z.reshape(-1, Ds)
    n_tiles = N_TOKENS // TN

    en, e_norm = pl.pallas_call(
        _normalize_embed_body,
        out_shape=(
            jax.ShapeDtypeStruct((1, N_EMBED), jnp.float32),
            jax.ShapeDtypeStruct((N_EMBED, D), jnp.float32),
        ),
    )(embed_weight)

    idx, d2min, acc = pl.pallas_call(
        _argmin_body,
        grid=(n_tiles,),
        in_specs=[
            pl.BlockSpec((TN, D), lambda i: (i, 0)),
            pl.BlockSpec((N_EMBED, D), lambda i: (0, 0)),
            pl.BlockSpec((1, N_EMBED), lambda i: (0, 0)),
        ],
        out_specs=(
            pl.BlockSpec((1, 1, TN), lambda i: (i, 0, 0)),
            pl.BlockSpec((1, 1, TN), lambda i: (i, 0, 0)),
            pl.BlockSpec((1, 1), lambda i: (0, 0)),
        ),
        out_shape=(
            jax.ShapeDtypeStruct((n_tiles, 1, TN), jnp.int32),
            jax.ShapeDtypeStruct((n_tiles, 1, TN), jnp.float32),
            jax.ShapeDtypeStruct((1, 1), jnp.float32),
        ),
    )(z_flat, e_norm, en)

    indices = idx.reshape(N_TOKENS)
    quantized = jnp.take(e_norm, indices, axis=0)
    loss = acc[0, 0] * ((1.0 + BETA) / (N_TOKENS * D))
    return (quantized.reshape(Bs, Ts, Ds), loss, indices.reshape(Bs, Ts))


# speedup vs baseline: 1.3281x; 1.1040x over previous
import functools

import jax
import jax.numpy as jnp
from jax import lax
from jax.experimental import pallas as pl
from jax.experimental.pallas import tpu as pltpu

N_EMBED = 8192
D = 256
BETA = 1.0
EPS = 1e-12

TN = 256
TK = 2048
N_TOKENS = 32768


def _normalize_embed_body(e_ref, en_out_ref, e_norm_ref):
    x = e_ref[...]
    n2 = jnp.sum(x * x, axis=1, keepdims=True)
    n = jnp.sqrt(n2)
    xn = x / jnp.maximum(n, EPS)
    e_norm_ref[...] = xn
    en_out_ref[...] = jnp.sum(xn * xn, axis=1, keepdims=True).reshape(1, N_EMBED)


def _argmin_body(z_ref, e_norm_ref, en_ref, idx_ref, d2_ref, acc_ref):
    i = pl.program_id(0)
    z = z_ref[...]
    zsq = jnp.sum(z * z, axis=1, keepdims=True)
    zn = z / jnp.maximum(jnp.sqrt(zsq), EPS)
    zn2 = jnp.sum(zn * zn, axis=1, keepdims=True)
    zm = -2.0 * zn

    best = jnp.full((TN, 1), jnp.inf, jnp.float32)
    bidx = jnp.zeros((TN, 1), jnp.float32)
    iota_f = lax.broadcasted_iota(jnp.int32, (TN, TK), 1).astype(jnp.float32)
    for kc in range(N_EMBED // TK):
        e_chunk = e_norm_ref[pl.ds(kc * TK, TK), :]
        en_chunk = en_ref[:, pl.ds(kc * TK, TK)]
        s2 = lax.dot_general(
            zm, e_chunk,
            dimension_numbers=(((1,), (1,)), ((), ())),
            preferred_element_type=jnp.float32,
        )
        a = zn2 + en_chunk
        d2 = a + s2
        m = jnp.min(d2, axis=1, keepdims=True)
        li = jnp.min(jnp.where(d2 == m, iota_f, 3e7), axis=1, keepdims=True)
        upd = m < best
        bidx = jnp.where(upd, li + kc * TK, bidx)
        best = jnp.where(upd, m, best)

    idx_ref[...] = bidx.astype(jnp.int32).reshape(1, 1, TN)
    d2_ref[...] = best.reshape(1, 1, TN)

    @pl.when(i == 0)
    def _():
        acc_ref[...] = jnp.zeros((1, 1), jnp.float32)

    acc_ref[...] += jnp.sum(best, keepdims=True)


@jax.jit
def kernel(z, embed_weight):
    Bs, Ts, Ds = z.shape
    z_flat = z.reshape(-1, Ds)
    n_tiles = N_TOKENS // TN

    en, e_norm = pl.pallas_call(
        _normalize_embed_body,
        out_shape=(
            jax.ShapeDtypeStruct((1, N_EMBED), jnp.float32),
            jax.ShapeDtypeStruct((N_EMBED, D), jnp.float32),
        ),
    )(embed_weight)

    idx, d2min, acc = pl.pallas_call(
        _argmin_body,
        grid=(n_tiles,),
        in_specs=[
            pl.BlockSpec((TN, D), lambda i: (i, 0)),
            pl.BlockSpec((N_EMBED, D), lambda i: (0, 0)),
            pl.BlockSpec((1, N_EMBED), lambda i: (0, 0)),
        ],
        out_specs=(
            pl.BlockSpec((1, 1, TN), lambda i: (i, 0, 0)),
            pl.BlockSpec((1, 1, TN), lambda i: (i, 0, 0)),
            pl.BlockSpec((1, 1), lambda i: (0, 0)),
        ),
        out_shape=(
            jax.ShapeDtypeStruct((n_tiles, 1, TN), jnp.int32),
            jax.ShapeDtypeStruct((n_tiles, 1, TN), jnp.float32),
            jax.ShapeDtypeStruct((1, 1), jnp.float32),
        ),
    )(z_flat, e_norm, en)

    indices = idx.reshape(N_TOKENS)
    quantized = jnp.take(e_norm, indices, axis=0)
    loss = acc[0, 0] * ((1.0 + BETA) / (N_TOKENS * D))
    return (quantized.reshape(Bs, Ts, Ds), loss, indices.reshape(Bs, Ts))


# speedup vs baseline: 1.3679x; 1.0300x over previous
import functools

import jax
import jax.numpy as jnp
from jax import lax
from jax.experimental import pallas as pl
from jax.experimental.pallas import tpu as pltpu

N_EMBED = 8192
D = 256
BETA = 1.0
EPS = 1e-12

TN = 256
TK = 2048
N_TOKENS = 32768


def _normalize_embed_body(e_ref, en_out_ref, e_norm_ref):
    x = e_ref[...]
    n2 = jnp.sum(x * x, axis=1, keepdims=True)
    n = jnp.sqrt(n2)
    xn = x / jnp.maximum(n, EPS)
    e_norm_ref[...] = xn
    en_out_ref[...] = jnp.sum(xn * xn, axis=1, keepdims=True).reshape(1, N_EMBED)


def _argmin_body(z_ref, e_norm_ref, en_ref, idx_ref, d2_ref):
    z = z_ref[...]
    zsq = jnp.sum(z * z, axis=1, keepdims=True)
    zn = z / jnp.maximum(jnp.sqrt(zsq), EPS)
    zn2 = jnp.sum(zn * zn, axis=1, keepdims=True)
    zm = -2.0 * zn

    best = jnp.full((TN, 1), jnp.inf, jnp.float32)
    bidx = jnp.zeros((TN, 1), jnp.float32)
    iota_f = lax.broadcasted_iota(jnp.int32, (TN, TK), 1).astype(jnp.float32)
    for kc in range(N_EMBED // TK):
        e_chunk = e_norm_ref[pl.ds(kc * TK, TK), :]
        en_chunk = en_ref[:, pl.ds(kc * TK, TK)]
        s2 = lax.dot_general(
            zm, e_chunk,
            dimension_numbers=(((1,), (1,)), ((), ())),
            preferred_element_type=jnp.float32,
        )
        a = zn2 + en_chunk
        d2 = a + s2
        m = jnp.min(d2, axis=1, keepdims=True)
        li = jnp.min(jnp.where(d2 == m, iota_f, 3e7), axis=1, keepdims=True)
        upd = m < best
        bidx = jnp.where(upd, li + kc * TK, bidx)
        best = jnp.where(upd, m, best)

    idx_ref[...] = bidx.astype(jnp.int32).reshape(1, 1, TN)
    d2_ref[...] = best.reshape(1, 1, TN)


@jax.jit
def kernel(z, embed_weight):
    Bs, Ts, Ds = z.shape
    z_flat = z.reshape(-1, Ds)
    n_tiles = N_TOKENS // TN

    en, e_norm = pl.pallas_call(
        _normalize_embed_body,
        out_shape=(
            jax.ShapeDtypeStruct((1, N_EMBED), jnp.float32),
            jax.ShapeDtypeStruct((N_EMBED, D), jnp.float32),
        ),
    )(embed_weight)

    idx, d2min = pl.pallas_call(
        _argmin_body,
        grid=(n_tiles,),
        in_specs=[
            pl.BlockSpec((TN, D), lambda i: (i, 0)),
            pl.BlockSpec((N_EMBED, D), lambda i: (0, 0)),
            pl.BlockSpec((1, N_EMBED), lambda i: (0, 0)),
        ],
        out_specs=(
            pl.BlockSpec((1, 1, TN), lambda i: (i, 0, 0)),
            pl.BlockSpec((1, 1, TN), lambda i: (i, 0, 0)),
        ),
        out_shape=(
            jax.ShapeDtypeStruct((n_tiles, 1, TN), jnp.int32),
            jax.ShapeDtypeStruct((n_tiles, 1, TN), jnp.float32),
        ),
        compiler_params=pltpu.CompilerParams(
            dimension_semantics=("parallel",),
        ),
    )(z_flat, e_norm, en)

    indices = idx.reshape(N_TOKENS)
    quantized = jnp.take(e_norm, indices, axis=0)
    loss = jnp.sum(d2min) * ((1.0 + BETA) / (N_TOKENS * D))
    return (quantized.reshape(Bs, Ts, Ds), loss, indices.reshape(Bs, Ts))


# speedup vs baseline: 1.6721x; 1.2224x over previous
import functools

import jax
import jax.numpy as jnp
from jax import lax
from jax.experimental import pallas as pl
from jax.experimental.pallas import tpu as pltpu
from jax.experimental.pallas import tpu_sc as plsc

N_EMBED = 8192
D = 256
BETA = 1.0
EPS = 1e-12

TN = 256
TK = 2048
N_TOKENS = 32768


def _normalize_embed_body(e_ref, en_out_ref, e_norm_ref):
    x = e_ref[...]
    n2 = jnp.sum(x * x, axis=1, keepdims=True)
    n = jnp.sqrt(n2)
    xn = x / jnp.maximum(n, EPS)
    e_norm_ref[...] = xn
    en_out_ref[...] = jnp.sum(xn * xn, axis=1, keepdims=True).reshape(1, N_EMBED)


def _argmin_body(z_ref, e_norm_ref, en_ref, idx_ref, d2_ref):
    z = z_ref[...]
    zsq = jnp.sum(z * z, axis=1, keepdims=True)
    zn = z / jnp.maximum(jnp.sqrt(zsq), EPS)
    zn2 = jnp.sum(zn * zn, axis=1, keepdims=True)
    zm = -2.0 * zn

    best = jnp.full((TN, 1), jnp.inf, jnp.float32)
    bidx = jnp.zeros((TN, 1), jnp.float32)
    iota_f = lax.broadcasted_iota(jnp.int32, (TN, TK), 1).astype(jnp.float32)
    for kc in range(N_EMBED // TK):
        e_chunk = e_norm_ref[pl.ds(kc * TK, TK), :]
        en_chunk = en_ref[:, pl.ds(kc * TK, TK)]
        s2 = lax.dot_general(
            zm, e_chunk,
            dimension_numbers=(((1,), (1,)), ((), ())),
            preferred_element_type=jnp.float32,
        )
        a = zn2 + en_chunk
        d2 = a + s2
        m = jnp.min(d2, axis=1, keepdims=True)
        li = jnp.min(jnp.where(d2 == m, iota_f, 3e7), axis=1, keepdims=True)
        upd = m < best
        bidx = jnp.where(upd, li + kc * TK, bidx)
        best = jnp.where(upd, m, best)

    idx_ref[...] = bidx.astype(jnp.int32).reshape(1, 1, TN)
    d2_ref[...] = best.reshape(1, 1, TN)


_SC_INFO = plsc.get_sparse_core_info()
_NC = _SC_INFO.num_cores
_NS = _SC_INFO.num_subcores
_NW = _NC * _NS
_ROWS_PER_W = N_TOKENS // _NW
_CH = 128
_NCH = _ROWS_PER_W // _CH


def _sc_gather_body(table_hbm, idx_hbm, out_hbm,
                    idx_v0, idx_v1, rows_v0, rows_v1, sem0, sem1):
    wid = lax.axis_index("s") * _NC + lax.axis_index("c")
    base = wid * _ROWS_PER_W
    idx_bufs = (idx_v0, idx_v1)
    row_bufs = (rows_v0, rows_v1)
    sems = (sem0, sem1)

    pltpu.sync_copy(idx_hbm.at[pl.ds(base, _CH)], idx_v0)
    cps = [pltpu.async_copy(table_hbm.at[idx_v0], rows_v0, sem0)]
    for c in range(_NCH):
        b = (c + 1) % 2
        if c + 1 < _NCH:
            pltpu.sync_copy(idx_hbm.at[pl.ds(base + (c + 1) * _CH, _CH)],
                            idx_bufs[b])
            cps.append(pltpu.async_copy(table_hbm.at[idx_bufs[b]],
                                        row_bufs[b], sems[b]))
        cps.pop(0).wait()
        pltpu.sync_copy(row_bufs[c % 2], out_hbm.at[pl.ds(base + c * _CH, _CH)])


_sc_gather = functools.partial(
    pl.kernel,
    out_type=jax.ShapeDtypeStruct((N_TOKENS, D), jnp.float32),
    mesh=plsc.VectorSubcoreMesh(core_axis_name="c", subcore_axis_name="s"),
    scratch_types=[
        pltpu.VMEM((_CH,), jnp.int32),
        pltpu.VMEM((_CH,), jnp.int32),
        pltpu.VMEM((_CH, D), jnp.float32),
        pltpu.VMEM((_CH, D), jnp.float32),
        pltpu.SemaphoreType.DMA,
        pltpu.SemaphoreType.DMA,
    ],
)(_sc_gather_body)


@jax.jit
def kernel(z, embed_weight):
    Bs, Ts, Ds = z.shape
    z_flat = z.reshape(-1, Ds)
    n_tiles = N_TOKENS // TN

    en, e_norm = pl.pallas_call(
        _normalize_embed_body,
        out_shape=(
            jax.ShapeDtypeStruct((1, N_EMBED), jnp.float32),
            jax.ShapeDtypeStruct((N_EMBED, D), jnp.float32),
        ),
    )(embed_weight)

    idx, d2min = pl.pallas_call(
        _argmin_body,
        grid=(n_tiles,),
        in_specs=[
            pl.BlockSpec((TN, D), lambda i: (i, 0)),
            pl.BlockSpec((N_EMBED, D), lambda i: (0, 0)),
            pl.BlockSpec((1, N_EMBED), lambda i: (0, 0)),
        ],
        out_specs=(
            pl.BlockSpec((1, 1, TN), lambda i: (i, 0, 0)),
            pl.BlockSpec((1, 1, TN), lambda i: (i, 0, 0)),
        ),
        out_shape=(
            jax.ShapeDtypeStruct((n_tiles, 1, TN), jnp.int32),
            jax.ShapeDtypeStruct((n_tiles, 1, TN), jnp.float32),
        ),
        compiler_params=pltpu.CompilerParams(
            dimension_semantics=("parallel",),
        ),
    )(z_flat, e_norm, en)

    indices = idx.reshape(N_TOKENS)
    quantized = _sc_gather(e_norm, indices)
    loss = jnp.sum(d2min) * ((1.0 + BETA) / (N_TOKENS * D))
    return (quantized.reshape(Bs, Ts, Ds), loss, indices.reshape(Bs, Ts))


# speedup vs baseline: 1.8300x; 1.0945x over previous
import functools

import jax
import jax.numpy as jnp
from jax import lax
from jax.experimental import pallas as pl
from jax.experimental.pallas import tpu as pltpu
from jax.experimental.pallas import tpu_sc as plsc

N_EMBED = 8192
D = 256
BETA = 1.0
EPS = 1e-12

TN = 256
TK = 2048
N_TOKENS = 32768


def _normalize_embed_body(e_ref, en_out_ref, e_norm_ref):
    x = e_ref[...]
    n2 = jnp.sum(x * x, axis=1, keepdims=True)
    n = jnp.sqrt(n2)
    xn = x / jnp.maximum(n, EPS)
    e_norm_ref[...] = xn
    en_out_ref[...] = jnp.sum(xn * xn, axis=1, keepdims=True).reshape(1, N_EMBED)


def _argmin_body(z_ref, e_norm_ref, en_ref, idx_ref, d2_ref):
    z = z_ref[...]
    zsq = jnp.sum(z * z, axis=1, keepdims=True)
    zn = z / jnp.maximum(jnp.sqrt(zsq), EPS)
    zn2 = jnp.sum(zn * zn, axis=1, keepdims=True)
    zm = -2.0 * zn

    best = jnp.full((TN, 1), jnp.inf, jnp.float32)
    bidx = jnp.zeros((TN, 1), jnp.float32)
    iota_f = lax.broadcasted_iota(jnp.int32, (TN, TK), 1).astype(jnp.float32)
    for kc in range(N_EMBED // TK):
        e_chunk = e_norm_ref[pl.ds(kc * TK, TK), :]
        en_chunk = en_ref[:, pl.ds(kc * TK, TK)]
        s2 = lax.dot_general(
            zm, e_chunk,
            dimension_numbers=(((1,), (1,)), ((), ())),
            preferred_element_type=jnp.float32,
        )
        a = zn2 + en_chunk
        d2 = a + s2
        m = jnp.min(d2, axis=1, keepdims=True)
        li = jnp.argmin(d2, axis=1).astype(jnp.float32).reshape(TN, 1)
        upd = m < best
        bidx = jnp.where(upd, li + kc * TK, bidx)
        best = jnp.where(upd, m, best)

    idx_ref[...] = bidx.astype(jnp.int32).reshape(1, 1, TN)
    d2_ref[...] = best.reshape(1, 1, TN)


_SC_INFO = plsc.get_sparse_core_info()
_NC = _SC_INFO.num_cores
_NS = _SC_INFO.num_subcores
_NW = _NC * _NS
_ROWS_PER_W = N_TOKENS // _NW
_CH = 128
_NCH = _ROWS_PER_W // _CH


def _sc_gather_body(table_hbm, idx_hbm, out_hbm,
                    idx_v0, idx_v1, rows_v0, rows_v1, sem0, sem1):
    wid = lax.axis_index("s") * _NC + lax.axis_index("c")
    base = wid * _ROWS_PER_W
    idx_bufs = (idx_v0, idx_v1)
    row_bufs = (rows_v0, rows_v1)
    sems = (sem0, sem1)

    pltpu.sync_copy(idx_hbm.at[pl.ds(base, _CH)], idx_v0)
    cps = [pltpu.async_copy(table_hbm.at[idx_v0], rows_v0, sem0)]
    for c in range(_NCH):
        b = (c + 1) % 2
        if c + 1 < _NCH:
            pltpu.sync_copy(idx_hbm.at[pl.ds(base + (c + 1) * _CH, _CH)],
                            idx_bufs[b])
            cps.append(pltpu.async_copy(table_hbm.at[idx_bufs[b]],
                                        row_bufs[b], sems[b]))
        cps.pop(0).wait()
        pltpu.sync_copy(row_bufs[c % 2], out_hbm.at[pl.ds(base + c * _CH, _CH)])


_sc_gather = functools.partial(
    pl.kernel,
    out_type=jax.ShapeDtypeStruct((N_TOKENS, D), jnp.float32),
    mesh=plsc.VectorSubcoreMesh(core_axis_name="c", subcore_axis_name="s"),
    scratch_types=[
        pltpu.VMEM((_CH,), jnp.int32),
        pltpu.VMEM((_CH,), jnp.int32),
        pltpu.VMEM((_CH, D), jnp.float32),
        pltpu.VMEM((_CH, D), jnp.float32),
        pltpu.SemaphoreType.DMA,
        pltpu.SemaphoreType.DMA,
    ],
)(_sc_gather_body)


@jax.jit
def kernel(z, embed_weight):
    Bs, Ts, Ds = z.shape
    z_flat = z.reshape(-1, Ds)
    n_tiles = N_TOKENS // TN

    en, e_norm = pl.pallas_call(
        _normalize_embed_body,
        out_shape=(
            jax.ShapeDtypeStruct((1, N_EMBED), jnp.float32),
            jax.ShapeDtypeStruct((N_EMBED, D), jnp.float32),
        ),
    )(embed_weight)

    idx, d2min = pl.pallas_call(
        _argmin_body,
        grid=(n_tiles,),
        in_specs=[
            pl.BlockSpec((TN, D), lambda i: (i, 0)),
            pl.BlockSpec((N_EMBED, D), lambda i: (0, 0)),
            pl.BlockSpec((1, N_EMBED), lambda i: (0, 0)),
        ],
        out_specs=(
            pl.BlockSpec((1, 1, TN), lambda i: (i, 0, 0)),
            pl.BlockSpec((1, 1, TN), lambda i: (i, 0, 0)),
        ),
        out_shape=(
            jax.ShapeDtypeStruct((n_tiles, 1, TN), jnp.int32),
            jax.ShapeDtypeStruct((n_tiles, 1, TN), jnp.float32),
        ),
        compiler_params=pltpu.CompilerParams(
            dimension_semantics=("parallel",),
        ),
    )(z_flat, e_norm, en)

    indices = idx.reshape(N_TOKENS)
    quantized = _sc_gather(e_norm, indices)
    loss = jnp.sum(d2min) * ((1.0 + BETA) / (N_TOKENS * D))
    return (quantized.reshape(Bs, Ts, Ds), loss, indices.reshape(Bs, Ts))


# speedup vs baseline: 1.8441x; 1.0077x over previous
import functools

import jax
import jax.numpy as jnp
from jax import lax
from jax.experimental import pallas as pl
from jax.experimental.pallas import tpu as pltpu
from jax.experimental.pallas import tpu_sc as plsc

N_EMBED = 8192
D = 256
BETA = 1.0
EPS = 1e-12

TN = 512
TK = 2048
N_TOKENS = 32768


def _normalize_embed_body(e_ref, en_out_ref, e_norm_ref):
    x = e_ref[...]
    n2 = jnp.sum(x * x, axis=1, keepdims=True)
    n = jnp.sqrt(n2)
    xn = x / jnp.maximum(n, EPS)
    e_norm_ref[...] = xn
    en_out_ref[...] = jnp.sum(xn * xn, axis=1, keepdims=True).reshape(1, N_EMBED)


def _argmin_body(z_ref, e_norm_ref, en_ref, idx_ref, d2_ref):
    z = z_ref[...]
    zsq = jnp.sum(z * z, axis=1, keepdims=True)
    zn = z / jnp.maximum(jnp.sqrt(zsq), EPS)
    zn2 = jnp.sum(zn * zn, axis=1, keepdims=True)
    zm = -2.0 * zn

    best = jnp.full((TN, 1), jnp.inf, jnp.float32)
    bidx = jnp.zeros((TN, 1), jnp.float32)
    iota_f = lax.broadcasted_iota(jnp.int32, (TN, TK), 1).astype(jnp.float32)
    for kc in range(N_EMBED // TK):
        e_chunk = e_norm_ref[pl.ds(kc * TK, TK), :]
        en_chunk = en_ref[:, pl.ds(kc * TK, TK)]
        s2 = lax.dot_general(
            zm, e_chunk,
            dimension_numbers=(((1,), (1,)), ((), ())),
            preferred_element_type=jnp.float32,
        )
        a = zn2 + en_chunk
        d2 = a + s2
        m = jnp.min(d2, axis=1, keepdims=True)
        li = jnp.argmin(d2, axis=1).astype(jnp.float32).reshape(TN, 1)
        upd = m < best
        bidx = jnp.where(upd, li + kc * TK, bidx)
        best = jnp.where(upd, m, best)

    idx_ref[...] = bidx.astype(jnp.int32).reshape(1, 1, TN)
    d2_ref[...] = best.reshape(1, 1, TN)


_SC_INFO = plsc.get_sparse_core_info()
_NC = _SC_INFO.num_cores
_NS = _SC_INFO.num_subcores
_NW = _NC * _NS
_ROWS_PER_W = N_TOKENS // _NW
_CH = 128
_NCH = _ROWS_PER_W // _CH


def _sc_gather_body(table_hbm, idx_hbm, out_hbm,
                    idx_v0, idx_v1, rows_v0, rows_v1, sem0, sem1):
    wid = lax.axis_index("s") * _NC + lax.axis_index("c")
    base = wid * _ROWS_PER_W
    idx_bufs = (idx_v0, idx_v1)
    row_bufs = (rows_v0, rows_v1)
    sems = (sem0, sem1)

    pltpu.sync_copy(idx_hbm.at[pl.ds(base, _CH)], idx_v0)
    cps = [pltpu.async_copy(table_hbm.at[idx_v0], rows_v0, sem0)]
    for c in range(_NCH):
        b = (c + 1) % 2
        if c + 1 < _NCH:
            pltpu.sync_copy(idx_hbm.at[pl.ds(base + (c + 1) * _CH, _CH)],
                            idx_bufs[b])
            cps.append(pltpu.async_copy(table_hbm.at[idx_bufs[b]],
                                        row_bufs[b], sems[b]))
        cps.pop(0).wait()
        pltpu.sync_copy(row_bufs[c % 2], out_hbm.at[pl.ds(base + c * _CH, _CH)])


_sc_gather = functools.partial(
    pl.kernel,
    out_type=jax.ShapeDtypeStruct((N_TOKENS, D), jnp.float32),
    mesh=plsc.VectorSubcoreMesh(core_axis_name="c", subcore_axis_name="s"),
    scratch_types=[
        pltpu.VMEM((_CH,), jnp.int32),
        pltpu.VMEM((_CH,), jnp.int32),
        pltpu.VMEM((_CH, D), jnp.float32),
        pltpu.VMEM((_CH, D), jnp.float32),
        pltpu.SemaphoreType.DMA,
        pltpu.SemaphoreType.DMA,
    ],
)(_sc_gather_body)


@jax.jit
def kernel(z, embed_weight):
    Bs, Ts, Ds = z.shape
    z_flat = z.reshape(-1, Ds)
    n_tiles = N_TOKENS // TN

    en, e_norm = pl.pallas_call(
        _normalize_embed_body,
        out_shape=(
            jax.ShapeDtypeStruct((1, N_EMBED), jnp.float32),
            jax.ShapeDtypeStruct((N_EMBED, D), jnp.float32),
        ),
    )(embed_weight)

    idx, d2min = pl.pallas_call(
        _argmin_body,
        grid=(n_tiles,),
        in_specs=[
            pl.BlockSpec((TN, D), lambda i: (i, 0)),
            pl.BlockSpec((N_EMBED, D), lambda i: (0, 0)),
            pl.BlockSpec((1, N_EMBED), lambda i: (0, 0)),
        ],
        out_specs=(
            pl.BlockSpec((1, 1, TN), lambda i: (i, 0, 0)),
            pl.BlockSpec((1, 1, TN), lambda i: (i, 0, 0)),
        ),
        out_shape=(
            jax.ShapeDtypeStruct((n_tiles, 1, TN), jnp.int32),
            jax.ShapeDtypeStruct((n_tiles, 1, TN), jnp.float32),
        ),
        compiler_params=pltpu.CompilerParams(
            dimension_semantics=("parallel",),
        ),
    )(z_flat, e_norm, en)

    indices = idx.reshape(N_TOKENS)
    quantized = _sc_gather(e_norm, indices)
    loss = jnp.sum(d2min) * ((1.0 + BETA) / (N_TOKENS * D))
    return (quantized.reshape(Bs, Ts, Ds), loss, indices.reshape(Bs, Ts))


# speedup vs baseline: 1.9728x; 1.0698x over previous
import functools

import jax
import jax.numpy as jnp
from jax import lax
from jax.experimental import pallas as pl
from jax.experimental.pallas import tpu as pltpu
from jax.experimental.pallas import tpu_sc as plsc

N_EMBED = 8192
D = 256
BETA = 1.0
EPS = 1e-12

TN = 512
TK = 4096
N_TOKENS = 32768


def _normalize_embed_body(e_ref, en_out_ref, e_norm_ref):
    x = e_ref[...]
    n2 = jnp.sum(x * x, axis=1, keepdims=True)
    n = jnp.sqrt(n2)
    xn = x / jnp.maximum(n, EPS)
    e_norm_ref[...] = xn
    en_out_ref[...] = jnp.sum(xn * xn, axis=1, keepdims=True).reshape(1, N_EMBED)


def _argmin_body(z_ref, e_norm_ref, en_ref, idx_ref, d2_ref):
    z = z_ref[...]
    zsq = jnp.sum(z * z, axis=1, keepdims=True)
    zn = z / jnp.maximum(jnp.sqrt(zsq), EPS)
    zn2 = jnp.sum(zn * zn, axis=1, keepdims=True)
    zm = -2.0 * zn

    best = jnp.full((TN, 1), jnp.inf, jnp.float32)
    bidx = jnp.zeros((TN, 1), jnp.float32)
    iota_f = lax.broadcasted_iota(jnp.int32, (TN, TK), 1).astype(jnp.float32)
    for kc in range(N_EMBED // TK):
        e_chunk = e_norm_ref[pl.ds(kc * TK, TK), :]
        en_chunk = en_ref[:, pl.ds(kc * TK, TK)]
        s2 = lax.dot_general(
            zm, e_chunk,
            dimension_numbers=(((1,), (1,)), ((), ())),
            preferred_element_type=jnp.float32,
        )
        a = zn2 + en_chunk
        d2 = a + s2
        m = jnp.min(d2, axis=1, keepdims=True)
        li = jnp.argmin(d2, axis=1).astype(jnp.float32).reshape(TN, 1)
        upd = m < best
        bidx = jnp.where(upd, li + kc * TK, bidx)
        best = jnp.where(upd, m, best)

    idx_ref[...] = bidx.astype(jnp.int32).reshape(1, 1, TN)
    d2_ref[...] = best.reshape(1, 1, TN)


_SC_INFO = plsc.get_sparse_core_info()
_NC = _SC_INFO.num_cores
_NS = _SC_INFO.num_subcores
_NW = _NC * _NS
_ROWS_PER_W = N_TOKENS // _NW
_CH = 128
_NCH = _ROWS_PER_W // _CH


def _sc_gather_body(table_hbm, idx_hbm, out_hbm,
                    idx_v0, idx_v1, rows_v0, rows_v1, sem0, sem1):
    wid = lax.axis_index("s") * _NC + lax.axis_index("c")
    base = wid * _ROWS_PER_W
    idx_bufs = (idx_v0, idx_v1)
    row_bufs = (rows_v0, rows_v1)
    sems = (sem0, sem1)

    pltpu.sync_copy(idx_hbm.at[pl.ds(base, _CH)], idx_v0)
    cps = [pltpu.async_copy(table_hbm.at[idx_v0], rows_v0, sem0)]
    for c in range(_NCH):
        b = (c + 1) % 2
        if c + 1 < _NCH:
            pltpu.sync_copy(idx_hbm.at[pl.ds(base + (c + 1) * _CH, _CH)],
                            idx_bufs[b])
            cps.append(pltpu.async_copy(table_hbm.at[idx_bufs[b]],
                                        row_bufs[b], sems[b]))
        cps.pop(0).wait()
        pltpu.sync_copy(row_bufs[c % 2], out_hbm.at[pl.ds(base + c * _CH, _CH)])


_sc_gather = functools.partial(
    pl.kernel,
    out_type=jax.ShapeDtypeStruct((N_TOKENS, D), jnp.float32),
    mesh=plsc.VectorSubcoreMesh(core_axis_name="c", subcore_axis_name="s"),
    scratch_types=[
        pltpu.VMEM((_CH,), jnp.int32),
        pltpu.VMEM((_CH,), jnp.int32),
        pltpu.VMEM((_CH, D), jnp.float32),
        pltpu.VMEM((_CH, D), jnp.float32),
        pltpu.SemaphoreType.DMA,
        pltpu.SemaphoreType.DMA,
    ],
)(_sc_gather_body)


@jax.jit
def kernel(z, embed_weight):
    Bs, Ts, Ds = z.shape
    z_flat = z.reshape(-1, Ds)
    n_tiles = N_TOKENS // TN

    en, e_norm = pl.pallas_call(
        _normalize_embed_body,
        out_shape=(
            jax.ShapeDtypeStruct((1, N_EMBED), jnp.float32),
            jax.ShapeDtypeStruct((N_EMBED, D), jnp.float32),
        ),
    )(embed_weight)

    idx, d2min = pl.pallas_call(
        _argmin_body,
        grid=(n_tiles,),
        in_specs=[
            pl.BlockSpec((TN, D), lambda i: (i, 0)),
            pl.BlockSpec((N_EMBED, D), lambda i: (0, 0)),
            pl.BlockSpec((1, N_EMBED), lambda i: (0, 0)),
        ],
        out_specs=(
            pl.BlockSpec((1, 1, TN), lambda i: (i, 0, 0)),
            pl.BlockSpec((1, 1, TN), lambda i: (i, 0, 0)),
        ),
        out_shape=(
            jax.ShapeDtypeStruct((n_tiles, 1, TN), jnp.int32),
            jax.ShapeDtypeStruct((n_tiles, 1, TN), jnp.float32),
        ),
        compiler_params=pltpu.CompilerParams(
            dimension_semantics=("parallel",),
        ),
    )(z_flat, e_norm, en)

    indices = idx.reshape(N_TOKENS)
    quantized = _sc_gather(e_norm, indices)
    loss = jnp.sum(d2min) * ((1.0 + BETA) / (N_TOKENS * D))
    return (quantized.reshape(Bs, Ts, Ds), loss, indices.reshape(Bs, Ts))


# speedup vs baseline: 1.9755x; 1.0014x over previous
import functools

import jax
import jax.numpy as jnp
from jax import lax
from jax.experimental import pallas as pl
from jax.experimental.pallas import tpu as pltpu
from jax.experimental.pallas import tpu_sc as plsc

N_EMBED = 8192
D = 256
BETA = 1.0
EPS = 1e-12

TN = 512
TK = 8192
N_TOKENS = 32768


def _normalize_embed_body(e_ref, en_out_ref, e_norm_ref):
    x = e_ref[...]
    n2 = jnp.sum(x * x, axis=1, keepdims=True)
    n = jnp.sqrt(n2)
    xn = x / jnp.maximum(n, EPS)
    e_norm_ref[...] = xn
    en_out_ref[...] = jnp.sum(xn * xn, axis=1, keepdims=True).reshape(1, N_EMBED)


def _argmin_body(z_ref, e_norm_ref, en_ref, idx_ref, d2_ref):
    z = z_ref[...]
    zsq = jnp.sum(z * z, axis=1, keepdims=True)
    zn = z / jnp.maximum(jnp.sqrt(zsq), EPS)
    zn2 = jnp.sum(zn * zn, axis=1, keepdims=True)
    zm = -2.0 * zn

    best = jnp.full((TN, 1), jnp.inf, jnp.float32)
    bidx = jnp.zeros((TN, 1), jnp.float32)
    iota_f = lax.broadcasted_iota(jnp.int32, (TN, TK), 1).astype(jnp.float32)
    for kc in range(N_EMBED // TK):
        e_chunk = e_norm_ref[pl.ds(kc * TK, TK), :]
        en_chunk = en_ref[:, pl.ds(kc * TK, TK)]
        s2 = lax.dot_general(
            zm, e_chunk,
            dimension_numbers=(((1,), (1,)), ((), ())),
            preferred_element_type=jnp.float32,
        )
        a = zn2 + en_chunk
        d2 = a + s2
        m = jnp.min(d2, axis=1, keepdims=True)
        li = jnp.argmin(d2, axis=1).astype(jnp.float32).reshape(TN, 1)
        upd = m < best
        bidx = jnp.where(upd, li + kc * TK, bidx)
        best = jnp.where(upd, m, best)

    idx_ref[...] = bidx.astype(jnp.int32).reshape(1, 1, TN)
    d2_ref[...] = best.reshape(1, 1, TN)


_SC_INFO = plsc.get_sparse_core_info()
_NC = _SC_INFO.num_cores
_NS = _SC_INFO.num_subcores
_NW = _NC * _NS
_ROWS_PER_W = N_TOKENS // _NW
_CH = 128
_NCH = _ROWS_PER_W // _CH


def _sc_gather_body(table_hbm, idx_hbm, out_hbm,
                    idx_v0, idx_v1, rows_v0, rows_v1, sem0, sem1):
    wid = lax.axis_index("s") * _NC + lax.axis_index("c")
    base = wid * _ROWS_PER_W
    idx_bufs = (idx_v0, idx_v1)
    row_bufs = (rows_v0, rows_v1)
    sems = (sem0, sem1)

    pltpu.sync_copy(idx_hbm.at[pl.ds(base, _CH)], idx_v0)
    cps = [pltpu.async_copy(table_hbm.at[idx_v0], rows_v0, sem0)]
    for c in range(_NCH):
        b = (c + 1) % 2
        if c + 1 < _NCH:
            pltpu.sync_copy(idx_hbm.at[pl.ds(base + (c + 1) * _CH, _CH)],
                            idx_bufs[b])
            cps.append(pltpu.async_copy(table_hbm.at[idx_bufs[b]],
                                        row_bufs[b], sems[b]))
        cps.pop(0).wait()
        pltpu.sync_copy(row_bufs[c % 2], out_hbm.at[pl.ds(base + c * _CH, _CH)])


_sc_gather = functools.partial(
    pl.kernel,
    out_type=jax.ShapeDtypeStruct((N_TOKENS, D), jnp.float32),
    mesh=plsc.VectorSubcoreMesh(core_axis_name="c", subcore_axis_name="s"),
    scratch_types=[
        pltpu.VMEM((_CH,), jnp.int32),
        pltpu.VMEM((_CH,), jnp.int32),
        pltpu.VMEM((_CH, D), jnp.float32),
        pltpu.VMEM((_CH, D), jnp.float32),
        pltpu.SemaphoreType.DMA,
        pltpu.SemaphoreType.DMA,
    ],
)(_sc_gather_body)


@jax.jit
def kernel(z, embed_weight):
    Bs, Ts, Ds = z.shape
    z_flat = z.reshape(-1, Ds)
    n_tiles = N_TOKENS // TN

    en, e_norm = pl.pallas_call(
        _normalize_embed_body,
        out_shape=(
            jax.ShapeDtypeStruct((1, N_EMBED), jnp.float32),
            jax.ShapeDtypeStruct((N_EMBED, D), jnp.float32),
        ),
    )(embed_weight)

    idx, d2min = pl.pallas_call(
        _argmin_body,
        grid=(n_tiles,),
        in_specs=[
            pl.BlockSpec((TN, D), lambda i: (i, 0)),
            pl.BlockSpec((N_EMBED, D), lambda i: (0, 0)),
            pl.BlockSpec((1, N_EMBED), lambda i: (0, 0)),
        ],
        out_specs=(
            pl.BlockSpec((1, 1, TN), lambda i: (i, 0, 0)),
            pl.BlockSpec((1, 1, TN), lambda i: (i, 0, 0)),
        ),
        out_shape=(
            jax.ShapeDtypeStruct((n_tiles, 1, TN), jnp.int32),
            jax.ShapeDtypeStruct((n_tiles, 1, TN), jnp.float32),
        ),
        compiler_params=pltpu.CompilerParams(
            dimension_semantics=("parallel",),
        ),
    )(z_flat, e_norm, en)

    indices = idx.reshape(N_TOKENS)
    quantized = _sc_gather(e_norm, indices)
    loss = jnp.sum(d2min) * ((1.0 + BETA) / (N_TOKENS * D))
    return (quantized.reshape(Bs, Ts, Ds), loss, indices.reshape(Bs, Ts))


# speedup vs baseline: 2.1891x; 1.1081x over previous
import functools

import jax
import jax.numpy as jnp
from jax import lax
from jax.experimental import pallas as pl
from jax.experimental.pallas import tpu as pltpu
from jax.experimental.pallas import tpu_sc as plsc

N_EMBED = 8192
D = 256
BETA = 1.0
EPS = 1e-12

TN = 512
TK = 8192
N_TOKENS = 32768


def _normalize_embed_body(e_ref, en_out_ref, e_norm_ref):
    x = e_ref[...]
    n2 = jnp.sum(x * x, axis=1, keepdims=True)
    n = jnp.sqrt(n2)
    xn = x / jnp.maximum(n, EPS)
    e_norm_ref[...] = xn
    en_out_ref[...] = jnp.sum(xn * xn, axis=1, keepdims=True).reshape(1, N_EMBED)


def _argmin_body(z_ref, e_norm_ref, en_ref, idx_ref, d2_ref):
    z = z_ref[...]
    zsq = jnp.sum(z * z, axis=1, keepdims=True)
    zn = z / jnp.maximum(jnp.sqrt(zsq), EPS)
    zn2 = jnp.sum(zn * zn, axis=1, keepdims=True)
    zm = -2.0 * zn

    val = jnp.full((TN, 128), jnp.inf, jnp.float32)
    cid = jnp.zeros((TN, 128), jnp.float32)
    for kc in range(N_EMBED // TK):
        e_chunk = e_norm_ref[pl.ds(kc * TK, TK), :]
        s2 = lax.dot_general(
            zm, e_chunk,
            dimension_numbers=(((1,), (1,)), ((), ())),
            preferred_element_type=jnp.float32,
        )
        for c in range(TK // 128):
            k0 = kc * TK + c * 128
            en_col = en_ref[:, pl.ds(k0, 128)]
            d2c = (zn2 + en_col) + lax.slice(s2, (0, c * 128), (TN, c * 128 + 128))
            lt = d2c < val
            val = jnp.where(lt, d2c, val)
            cid = jnp.where(lt, jnp.float32(k0 // 128), cid)

    m = jnp.min(val, axis=1, keepdims=True)
    lane = lax.broadcasted_iota(jnp.int32, (TN, 128), 1).astype(jnp.float32)
    k_cand = cid * 128.0 + lane
    li = jnp.min(jnp.where(val == m, k_cand, 3e7), axis=1, keepdims=True)

    idx_ref[...] = li.astype(jnp.int32).reshape(1, 1, TN)
    d2_ref[...] = m.reshape(1, 1, TN)


_SC_INFO = plsc.get_sparse_core_info()
_NC = _SC_INFO.num_cores
_NS = _SC_INFO.num_subcores
_NW = _NC * _NS
_ROWS_PER_W = N_TOKENS // _NW
_CH = 128
_NCH = _ROWS_PER_W // _CH


def _sc_gather_body(table_hbm, idx_hbm, out_hbm,
                    idx_v0, idx_v1, rows_v0, rows_v1, sem0, sem1):
    wid = lax.axis_index("s") * _NC + lax.axis_index("c")
    base = wid * _ROWS_PER_W
    idx_bufs = (idx_v0, idx_v1)
    row_bufs = (rows_v0, rows_v1)
    sems = (sem0, sem1)

    pltpu.sync_copy(idx_hbm.at[pl.ds(base, _CH)], idx_v0)
    cps = [pltpu.async_copy(table_hbm.at[idx_v0], rows_v0, sem0)]
    for c in range(_NCH):
        b = (c + 1) % 2
        if c + 1 < _NCH:
            pltpu.sync_copy(idx_hbm.at[pl.ds(base + (c + 1) * _CH, _CH)],
                            idx_bufs[b])
            cps.append(pltpu.async_copy(table_hbm.at[idx_bufs[b]],
                                        row_bufs[b], sems[b]))
        cps.pop(0).wait()
        pltpu.sync_copy(row_bufs[c % 2], out_hbm.at[pl.ds(base + c * _CH, _CH)])


_sc_gather = functools.partial(
    pl.kernel,
    out_type=jax.ShapeDtypeStruct((N_TOKENS, D), jnp.float32),
    mesh=plsc.VectorSubcoreMesh(core_axis_name="c", subcore_axis_name="s"),
    scratch_types=[
        pltpu.VMEM((_CH,), jnp.int32),
        pltpu.VMEM((_CH,), jnp.int32),
        pltpu.VMEM((_CH, D), jnp.float32),
        pltpu.VMEM((_CH, D), jnp.float32),
        pltpu.SemaphoreType.DMA,
        pltpu.SemaphoreType.DMA,
    ],
)(_sc_gather_body)


@jax.jit
def kernel(z, embed_weight):
    Bs, Ts, Ds = z.shape
    z_flat = z.reshape(-1, Ds)
    n_tiles = N_TOKENS // TN

    en, e_norm = pl.pallas_call(
        _normalize_embed_body,
        out_shape=(
            jax.ShapeDtypeStruct((1, N_EMBED), jnp.float32),
            jax.ShapeDtypeStruct((N_EMBED, D), jnp.float32),
        ),
    )(embed_weight)

    idx, d2min = pl.pallas_call(
        _argmin_body,
        grid=(n_tiles,),
        in_specs=[
            pl.BlockSpec((TN, D), lambda i: (i, 0)),
            pl.BlockSpec((N_EMBED, D), lambda i: (0, 0)),
            pl.BlockSpec((1, N_EMBED), lambda i: (0, 0)),
        ],
        out_specs=(
            pl.BlockSpec((1, 1, TN), lambda i: (i, 0, 0)),
            pl.BlockSpec((1, 1, TN), lambda i: (i, 0, 0)),
        ),
        out_shape=(
            jax.ShapeDtypeStruct((n_tiles, 1, TN), jnp.int32),
            jax.ShapeDtypeStruct((n_tiles, 1, TN), jnp.float32),
        ),
        compiler_params=pltpu.CompilerParams(
            dimension_semantics=("parallel",),
        ),
    )(z_flat, e_norm, en)

    indices = idx.reshape(N_TOKENS)
    quantized = _sc_gather(e_norm, indices)
    loss = jnp.sum(d2min) * ((1.0 + BETA) / (N_TOKENS * D))
    return (quantized.reshape(Bs, Ts, Ds), loss, indices.reshape(Bs, Ts))
